# baseline (speedup 1.0000x reference)
import jax
import jax.numpy as jnp
from jax.experimental import pallas as pl

_BC = 64


def _copy_kernel(h_ref, o_ref):
    o_ref[...] = h_ref[...]


def kernel(encoded_sents, indices, hiddens, keys, U, V, W):
    B, N, D = hiddens.shape
    h2 = hiddens.reshape(B, 250, 128)
    out = pl.pallas_call(
        _copy_kernel,
        grid=(B // _BC,),
        in_specs=[pl.BlockSpec((_BC, 250, 128), lambda b: (b, 0, 0))],
        out_specs=pl.BlockSpec((_BC, 250, 128), lambda b: (b, 0, 0)),
        out_shape=jax.ShapeDtypeStruct((B, 250, 128), jnp.float32),
    )(h2)
    return out.reshape(B, N, D)


# speedup vs baseline: 3.1921x; 3.1921x over previous
import functools

import jax
import jax.numpy as jnp
from jax.experimental import pallas as pl
from jax.experimental.pallas import tpu as pltpu

_NC = 50


def _update_kernel(idx_ref, prev_ref, h_blk, k_blk, s_ref, u_ref, v_ref,
                   w_ref, upd_ref):
    t = pl.program_id(0)
    pt = prev_ref[t]

    h0 = h_blk[0]
    k_i = k_blk[0]
    s = s_ref[...]

    hc = upd_ref[pl.ds(jnp.maximum(pt, 0), 1)][0]
    h_i = jnp.where(pt >= 0, hc, h0)

    g = jax.nn.sigmoid(jnp.sum(s * (h_i + k_i), axis=0, keepdims=True))
    h_tilde = jnp.maximum(
        jnp.dot(u_ref[...], h_i, preferred_element_type=jnp.float32)
        + jnp.dot(v_ref[...], k_i, preferred_element_type=jnp.float32)
        + jnp.dot(w_ref[...], s, preferred_element_type=jnp.float32),
        0.0,
    )
    h_new = h_i + g * h_tilde
    norm = jnp.sqrt(jnp.maximum(jnp.sum(h_new * h_new, axis=0, keepdims=True),
                                1e-12))
    upd_ref[pl.ds(t, 1)] = (h_new / norm)[None]


def _splice_kernel(T, idx_ref, last_ref, h_ref, upd_ref, out_ref):
    out_ref[...] = h_ref[...]
    n0 = pl.program_id(0) * _NC
    for t in range(T):
        row = idx_ref[t] - n0
        @pl.when((last_ref[t] != 0) & (row >= 0) & (row < _NC))
        def _():
            out_ref[pl.ds(row, 1)] = upd_ref[pl.ds(t, 1)]


@jax.jit
def kernel(encoded_sents, indices, hiddens, keys, U, V, W):
    B, N, D = hiddens.shape
    T = indices.shape[0]
    indices = indices.astype(jnp.int32)

    ht = jnp.transpose(hiddens, (1, 2, 0))
    kt = jnp.transpose(keys, (1, 2, 0))
    st = encoded_sents.T

    eq = indices[:, None] == indices[None, :]
    steps = jnp.arange(T, dtype=jnp.int32)
    prev = jnp.max(jnp.where(jnp.tril(eq, k=-1), steps[None, :], -1), axis=1)
    last = (jnp.sum(jnp.triu(eq, k=1), axis=1) == 0).astype(jnp.int32)

    upd = pl.pallas_call(
        _update_kernel,
        grid_spec=pltpu.PrefetchScalarGridSpec(
            num_scalar_prefetch=2,
            grid=(T,),
            in_specs=[
                pl.BlockSpec((1, D, B), lambda t, idx, prv: (idx[t], 0, 0)),
                pl.BlockSpec((1, D, B), lambda t, idx, prv: (idx[t], 0, 0)),
                pl.BlockSpec((D, B), lambda t, idx, prv: (0, 0)),
                pl.BlockSpec((D, D), lambda t, idx, prv: (0, 0)),
                pl.BlockSpec((D, D), lambda t, idx, prv: (0, 0)),
                pl.BlockSpec((D, D), lambda t, idx, prv: (0, 0)),
            ],
            out_specs=pl.BlockSpec((T, D, B), lambda t, idx, prv: (0, 0, 0)),
        ),
        out_shape=jax.ShapeDtypeStruct((T, D, B), jnp.float32),
    )(indices, prev, ht, kt, st, U, V, W)

    out_t = pl.pallas_call(
        functools.partial(_splice_kernel, T),
        grid_spec=pltpu.PrefetchScalarGridSpec(
            num_scalar_prefetch=2,
            grid=(N // _NC,),
            in_specs=[
                pl.BlockSpec((_NC, D, B), lambda n, idx, lst: (n, 0, 0)),
                pl.BlockSpec((T, D, B), lambda n, idx, lst: (0, 0, 0)),
            ],
            out_specs=pl.BlockSpec((_NC, D, B), lambda n, idx, lst: (n, 0, 0)),
        ),
        out_shape=jax.ShapeDtypeStruct((N, D, B), jnp.float32),
    )(indices, last, ht, upd)
    return jnp.transpose(out_t, (2, 0, 1))


# speedup vs baseline: 3.8243x; 1.1980x over previous
import functools

import jax
import jax.numpy as jnp
from jax.experimental import pallas as pl
from jax.experimental.pallas import tpu as pltpu

_NC = 50


def _stream_kernel(T, NC, idx_ref, h_ref, s_ref, u_ref, v_ref, w_ref,
                   *k_refs_and_out):
    k_refs = k_refs_and_out[:T]
    out_ref = k_refs_and_out[T]
    out_ref[...] = h_ref[...]
    n0 = pl.program_id(0) * NC
    s = s_ref[...]
    for t in range(T):
        row = idx_ref[t] - n0

        @pl.when((row >= 0) & (row < NC))
        def _():
            h_i = out_ref[pl.ds(row, 1)][0]
            k_i = k_refs[t][0]
            g = jax.nn.sigmoid(jnp.sum(s * (h_i + k_i), axis=0,
                                       keepdims=True))
            h_tilde = jnp.maximum(
                jnp.dot(u_ref[...], h_i, preferred_element_type=jnp.float32)
                + jnp.dot(v_ref[...], k_i, preferred_element_type=jnp.float32)
                + jnp.dot(w_ref[...], s, preferred_element_type=jnp.float32),
                0.0,
            )
            h_new = h_i + g * h_tilde
            norm = jnp.sqrt(jnp.maximum(
                jnp.sum(h_new * h_new, axis=0, keepdims=True), 1e-12))
            out_ref[pl.ds(row, 1)] = (h_new / norm)[None]


@jax.jit
def kernel(encoded_sents, indices, hiddens, keys, U, V, W):
    B, N, D = hiddens.shape
    T = indices.shape[0]
    indices = indices.astype(jnp.int32)

    ht = jnp.transpose(hiddens, (1, 2, 0))
    kt = jnp.transpose(keys, (1, 2, 0))
    st = encoded_sents.T

    def k_spec(t):
        return pl.BlockSpec((1, D, B), lambda n, idx, _t=t: (idx[_t], 0, 0))

    out_t = pl.pallas_call(
        functools.partial(_stream_kernel, T, _NC),
        grid_spec=pltpu.PrefetchScalarGridSpec(
            num_scalar_prefetch=1,
            grid=(N // _NC,),
            in_specs=[
                pl.BlockSpec((_NC, D, B), lambda n, idx: (n, 0, 0)),
                pl.BlockSpec((D, B), lambda n, idx: (0, 0)),
                pl.BlockSpec((D, D), lambda n, idx: (0, 0)),
                pl.BlockSpec((D, D), lambda n, idx: (0, 0)),
                pl.BlockSpec((D, D), lambda n, idx: (0, 0)),
            ] + [k_spec(t) for t in range(T)],
            out_specs=pl.BlockSpec((_NC, D, B), lambda n, idx: (n, 0, 0)),
        ),
        out_shape=jax.ShapeDtypeStruct((N, D, B), jnp.float32),
    )(indices, ht, st, U, V, W, *([kt] * T))
    return jnp.transpose(out_t, (2, 0, 1))


# speedup vs baseline: 3.9321x; 1.0282x over previous
import functools

import jax
import jax.numpy as jnp
from jax.experimental import pallas as pl
from jax.experimental.pallas import tpu as pltpu

_NC = 100


def _stream_kernel(T, NC, idx_ref, h_ref, s_ref, u_ref, v_ref, w_ref,
                   *k_refs_and_out):
    k_refs = k_refs_and_out[:T]
    out_ref = k_refs_and_out[T]
    out_ref[...] = h_ref[...]
    n0 = pl.program_id(0) * NC
    s = s_ref[...]
    for t in range(T):
        row = idx_ref[t] - n0

        @pl.when((row >= 0) & (row < NC))
        def _():
            h_i = out_ref[pl.ds(row, 1)][0]
            k_i = k_refs[t][0]
            g = jax.nn.sigmoid(jnp.sum(s * (h_i + k_i), axis=0,
                                       keepdims=True))
            h_tilde = jnp.maximum(
                jnp.dot(u_ref[...], h_i, preferred_element_type=jnp.float32)
                + jnp.dot(v_ref[...], k_i, preferred_element_type=jnp.float32)
                + jnp.dot(w_ref[...], s, preferred_element_type=jnp.float32),
                0.0,
            )
            h_new = h_i + g * h_tilde
            norm = jnp.sqrt(jnp.maximum(
                jnp.sum(h_new * h_new, axis=0, keepdims=True), 1e-12))
            out_ref[pl.ds(row, 1)] = (h_new / norm)[None]


@jax.jit
def kernel(encoded_sents, indices, hiddens, keys, U, V, W):
    B, N, D = hiddens.shape
    T = indices.shape[0]
    indices = indices.astype(jnp.int32)

    ht = jnp.transpose(hiddens, (1, 2, 0))
    kt = jnp.transpose(keys, (1, 2, 0))
    st = encoded_sents.T

    def k_spec(t):
        return pl.BlockSpec((1, D, B), lambda n, idx, _t=t: (idx[_t], 0, 0))

    out_t = pl.pallas_call(
        functools.partial(_stream_kernel, T, _NC),
        grid_spec=pltpu.PrefetchScalarGridSpec(
            num_scalar_prefetch=1,
            grid=(N // _NC,),
            in_specs=[
                pl.BlockSpec((_NC, D, B), lambda n, idx: (n, 0, 0)),
                pl.BlockSpec((D, B), lambda n, idx: (0, 0)),
                pl.BlockSpec((D, D), lambda n, idx: (0, 0)),
                pl.BlockSpec((D, D), lambda n, idx: (0, 0)),
                pl.BlockSpec((D, D), lambda n, idx: (0, 0)),
            ] + [k_spec(t) for t in range(T)],
            out_specs=pl.BlockSpec((_NC, D, B), lambda n, idx: (n, 0, 0)),
        ),
        out_shape=jax.ShapeDtypeStruct((N, D, B), jnp.float32),
    )(indices, ht, st, U, V, W, *([kt] * T))
    return jnp.transpose(out_t, (2, 0, 1))
